# baseline (speedup 1.0000x reference)
import jax
import jax.numpy as jnp
from jax import lax
from jax.experimental import pallas as pl
from jax.experimental.pallas import tpu as pltpu
from jax.experimental.pallas import tpu_sc as plsc

N_NODES = 50000
NW = 32
NREL = 128
ROW_GS, ROW_GD, ROW_P5, ROW_P6, N_ROWS = 0, 72, 144, 184, 224
HTOT = N_ROWS * NREL
PTAB = 4240
CHUNKS = (12504, 12496)
CBUF = 12512


def _sc_hist_body(ei_hbm, typ_hbm, aux_hbm, out_hbm,
                  table_v, hist_v, ptab_v, aux_v, src_v, dst_v, typ_v):
  wid = lax.axis_index("s") * 2 + lax.axis_index("c")
  z16i = jnp.zeros((16,), jnp.int32)
  z16f = jnp.zeros((16,), jnp.float32)
  ones16 = jnp.ones((16,), jnp.float32)
  lanes = lax.iota(jnp.int32, 16)

  def zero_i(ref, n):
    def b(i, carry):
      ref[pl.ds(i * 16, 16)] = z16i
      return carry
    lax.fori_loop(0, n // 16, b, 0, unroll=8)

  def zero_f(ref, n):
    def b(i, carry):
      ref[pl.ds(i * 16, 16)] = z16f
      return carry
    lax.fori_loop(0, n // 16, b, 0, unroll=8)

  zero_i(table_v, N_NODES)
  zero_i(ptab_v, PTAB)
  zero_f(hist_v, HTOT)

  pltpu.sync_copy(aux_hbm, aux_v)

  for j in range(4):
    ids_j = aux_v[pl.ds(j * 16, 16)]
    cod_j = aux_v[pl.ds(64 + j * 16, 16)]
    fm_j = aux_v[pl.ds(192 + j * 16, 16)]
    plsc.addupdate_scatter(table_v, [ids_j], cod_j, mask=fm_j > 0)
  for j in range(2):
    pk_j = aux_v[pl.ds(128 + j * 16, 16)]
    pv_j = aux_v[pl.ds(160 + j * 16, 16)]
    pfm_j = aux_v[pl.ds(256 + j * 16, 16)]
    plsc.addupdate_scatter(ptab_v, [pk_j], pv_j, mask=pfm_j > 0)

  n_edges = ei_hbm.shape[0] // 2
  epw = n_edges // NW
  base = wid * epw

  def accumulate(s, d, t, msk):
    cs = plsc.load_gather(table_v, [s])
    cd = plsc.load_gather(table_v, [d])
    k5 = plsc.load_gather(ptab_v, [cs * 65 + cd])
    k6 = plsc.load_gather(ptab_v, [cd * 65 + cs])
    m_s, m_d = cs > 0, cd > 0
    m5, m6 = k5 > 0, k6 > 0
    if msk is not None:
      m_s, m_d = msk & m_s, msk & m_d
      m5, m6 = msk & m5, msk & m6
    plsc.addupdate_scatter(hist_v, [(ROW_GS + cs) * NREL + t], ones16, mask=m_s)
    plsc.addupdate_scatter(hist_v, [(ROW_GD + cd) * NREL + t], ones16, mask=m_d)
    plsc.addupdate_scatter(hist_v, [(ROW_P5 + k5) * NREL + t], ones16, mask=m5)
    plsc.addupdate_scatter(hist_v, [(ROW_P6 + k6) * NREL + t], ones16, mask=m6)

  def ebody(i):
    off = i * 16
    accumulate(src_v[pl.ds(off, 16)], dst_v[pl.ds(off, 16)],
               typ_v[pl.ds(off, 16)], None)

  coff = 0
  for clen in CHUNKS:
    pltpu.sync_copy(ei_hbm.at[pl.ds(base + coff, clen)],
                    src_v.at[pl.ds(0, clen)])
    pltpu.sync_copy(ei_hbm.at[pl.ds(n_edges + base + coff, clen)],
                    dst_v.at[pl.ds(0, clen)])
    pltpu.sync_copy(typ_hbm.at[pl.ds(base + coff, clen)],
                    typ_v.at[pl.ds(0, clen)])
    n_full = clen // 16
    plsc.parallel_loop(0, n_full, unroll=4)(ebody)
    rem = clen - n_full * 16
    if rem:
      moff = n_full * 16
      mt = lanes < rem
      accumulate(jnp.where(mt, src_v[pl.ds(moff, 16)], 0),
                 jnp.where(mt, dst_v[pl.ds(moff, 16)], 0),
                 jnp.where(mt, typ_v[pl.ds(moff, 16)], 0), mt)
    coff += clen

  pltpu.sync_copy(hist_v, out_hbm.at[pl.ds(wid * HTOT, HTOT)])


def _sc_hist(ei, etype, aux):
  call = pl.kernel(
      _sc_hist_body,
      out_type=jax.ShapeDtypeStruct((NW * HTOT,), jnp.float32),
      mesh=plsc.VectorSubcoreMesh(core_axis_name="c", subcore_axis_name="s"),
      compiler_params=pltpu.CompilerParams(needs_layout_passes=False),
      scratch_types=[
          pltpu.VMEM((N_NODES,), jnp.int32),
          pltpu.VMEM((HTOT,), jnp.float32),
          pltpu.VMEM((PTAB,), jnp.int32),
          pltpu.VMEM((288,), jnp.int32),
          pltpu.VMEM((CBUF,), jnp.int32),
          pltpu.VMEM((CBUF,), jnp.int32),
          pltpu.VMEM((CBUF,), jnp.int32),
      ],
  )
  return call(ei, etype, aux)


def _combine_body(hist_ref, meta_ref, rel_emb_ref, w6_ref, b6_ref, wr_ref,
                  br_ref, wf_ref, bf_ref, out_ref):
  f32 = jnp.float32
  H = hist_ref[0:N_ROWS, :]
  for k in range(1, NW):
    H = H + hist_ref[k * N_ROWS:(k + 1) * N_ROWS, :]
  Gs = H[ROW_GS:ROW_GS + 72]
  Gd = H[ROW_GD:ROW_GD + 72]
  P5 = H[ROW_P5:ROW_P5 + 40]
  P6 = H[ROW_P6:ROW_P6 + 40]
  Emb = rel_emb_ref[...]
  dot = lambda a, b: jnp.dot(a, b, preferred_element_type=f32)
  GsE, GdE, P5E, P6E = dot(Gs, Emb), dot(Gd, Emb), dot(P5, Emb), dot(P6, Emb)
  Gs_c = jnp.sum(Gs, axis=1, keepdims=True)
  Gd_c = jnp.sum(Gd, axis=1, keepdims=True)
  P5_c = jnp.sum(P5, axis=1, keepdims=True)
  P6_c = jnp.sum(P6, axis=1, keepdims=True)
  meta = meta_ref[...]
  B = meta.shape[1]
  ch = jnp.transpose(meta[0:1, :])
  ct = jnp.transpose(meta[1:2, :])
  pr = jnp.transpose(meta[2:3, :])
  rl = jnp.transpose(meta[3:4, :])
  Oh = (lax.broadcasted_iota(jnp.int32, (B, 72), 1) == ch).astype(f32)
  Ot = (lax.broadcasted_iota(jnp.int32, (B, 72), 1) == ct).astype(f32)
  Op = (lax.broadcasted_iota(jnp.int32, (B, 40), 1) == pr).astype(f32)
  Orl = (lax.broadcasted_iota(jnp.int32, (B, NREL), 1) == rl).astype(f32)
  S1, c1 = dot(Oh, GdE), dot(Oh, Gd_c)
  S2, c2 = dot(Oh, GsE), dot(Oh, Gs_c)
  S3, c3 = dot(Ot, GdE), dot(Ot, Gd_c)
  S4, c4 = dot(Ot, GsE), dot(Ot, Gs_c)
  M5, n5 = dot(Op, P5E), dot(Op, P5_c)
  M6, n6 = dot(Op, P6E), dot(Op, P6_c)
  W6 = w6_ref[...]
  b6 = b6_ref[...]
  cat1 = dot(S1 - M6, W6[0]) + (c1 - n6) * b6[0:1, :]
  cat2 = dot(S2 - M5, W6[1]) + (c2 - n5) * b6[1:2, :]
  cat3 = dot(S3 - M5, W6[2]) + (c3 - n5) * b6[2:3, :]
  cat4 = dot(S4 - M6, W6[3]) + (c4 - n6) * b6[3:4, :]
  cat5 = dot(M5, W6[4]) + n5 * b6[4:5, :]
  cat6 = dot(M6, W6[5]) + n6 * b6[5:6, :]
  agg = cat1 + cat2 + cat3 + cat4 + cat5 + cat6
  agg = jnp.where(agg > 0, agg, 0.2 * agg)
  tr = dot(dot(Orl, Emb), wr_ref[...]) + br_ref[...]
  rep = tr + agg
  out_ref[...] = dot(rep, wf_ref[...]) + bf_ref[...]


def kernel(edge_index, edge_type, head_ids, tail_ids, rel_labels,
           rel_emb, W6, b6, Wr, br, Wf, bf):
  B = head_ids.shape[0]
  ei = edge_index.astype(jnp.int32).reshape(-1)
  et = edge_type.astype(jnp.int32)

  ids = jnp.concatenate([head_ids, tail_ids]).astype(jnp.int32)
  eq = ids[:, None] == ids[None, :]
  codes = (jnp.argmax(eq, axis=1) + 1).astype(jnp.int32)
  ch, ct = codes[:B], codes[B:]
  pkey = ch * 65 + ct
  pfirst = jnp.argmax(pkey[:, None] == pkey[None, :], axis=1)
  pr = (pfirst + 1).astype(jnp.int32)
  nid = ids.shape[0]
  fmask = (jnp.argmax(eq, axis=1) == jnp.arange(nid)).astype(jnp.int32)
  pfmask = (pfirst == jnp.arange(B)).astype(jnp.int32)
  aux = jnp.concatenate([ids, codes, pkey, pr, fmask, pfmask]).astype(jnp.int32)

  hist = _sc_hist(ei, et, aux).reshape(NW * N_ROWS, NREL)
  meta = jnp.stack([ch, ct, pr, rel_labels.astype(jnp.int32)])

  scores = pl.pallas_call(
      _combine_body,
      out_shape=jax.ShapeDtypeStruct((B, 1), jnp.float32),
  )(hist, meta, rel_emb, W6, b6, Wr, br.reshape(1, -1), Wf, bf.reshape(1, 1))
  return scores


# speedup vs baseline: 1.1186x; 1.1186x over previous
import jax
import jax.numpy as jnp
from jax import lax
from jax.experimental import pallas as pl
from jax.experimental.pallas import tpu as pltpu
from jax.experimental.pallas import tpu_sc as plsc

N_NODES = 50000
NW = 32
NREL = 128
ROW_GS, ROW_GD, ROW_P5, ROW_P6, N_ROWS = 0, 72, 144, 184, 224
HTOT = N_ROWS * NREL
PTAB = 4240
CHUNKS = (6256, 6248, 6248, 6248)
CBUF = 6256


def _sc_hist_body(ei_hbm, typ_hbm, aux_hbm, out_hbm,
                  table_v, hist_v, ptab_v, aux_v,
                  src_v, dst_v, typ_v, sem):
  wid = lax.axis_index("s") * 2 + lax.axis_index("c")
  z16i = jnp.zeros((16,), jnp.int32)
  z16f = jnp.zeros((16,), jnp.float32)
  ones16 = jnp.ones((16,), jnp.float32)
  lanes = lax.iota(jnp.int32, 16)

  def zero_i(ref, n):
    def b(i, carry):
      ref[pl.ds(i * 16, 16)] = z16i
      return carry
    lax.fori_loop(0, n // 16, b, 0, unroll=8)

  def zero_f(ref, n):
    def b(i, carry):
      ref[pl.ds(i * 16, 16)] = z16f
      return carry
    lax.fori_loop(0, n // 16, b, 0, unroll=8)

  n_edges = ei_hbm.shape[0] // 2
  epw = n_edges // NW
  base = wid * epw
  n_chunks = len(CHUNKS)
  offs = [sum(CHUNKS[:c]) for c in range(n_chunks)]

  def start_chunk(c):
    b = c % 2
    o = base + offs[c]
    clen = CHUNKS[c]
    return (
        pltpu.async_copy(ei_hbm.at[pl.ds(o, clen)],
                         src_v.at[pl.ds(b * CBUF, clen)], sem),
        pltpu.async_copy(ei_hbm.at[pl.ds(n_edges + o, clen)],
                         dst_v.at[pl.ds(b * CBUF, clen)], sem),
        pltpu.async_copy(typ_hbm.at[pl.ds(o, clen)],
                         typ_v.at[pl.ds(b * CBUF, clen)], sem),
    )

  pending = [start_chunk(0), start_chunk(1)]

  zero_i(table_v, N_NODES)
  zero_i(ptab_v, PTAB)
  zero_f(hist_v, HTOT)

  pltpu.sync_copy(aux_hbm, aux_v)

  for j in range(4):
    ids_j = aux_v[pl.ds(j * 16, 16)]
    cod_j = aux_v[pl.ds(64 + j * 16, 16)]
    fm_j = aux_v[pl.ds(192 + j * 16, 16)]
    plsc.addupdate_scatter(table_v, [ids_j], cod_j, mask=fm_j > 0)
  for j in range(2):
    pk_j = aux_v[pl.ds(128 + j * 16, 16)]
    pv_j = aux_v[pl.ds(160 + j * 16, 16)]
    pfm_j = aux_v[pl.ds(256 + j * 16, 16)]
    plsc.addupdate_scatter(ptab_v, [pk_j], pv_j, mask=pfm_j > 0)

  def accumulate(s, d, t, msk):
    cs = plsc.load_gather(table_v, [s])
    cd = plsc.load_gather(table_v, [d])
    k5 = plsc.load_gather(ptab_v, [cs * 65 + cd])
    k6 = plsc.load_gather(ptab_v, [cd * 65 + cs])
    m_s, m_d = cs > 0, cd > 0
    m5, m6 = k5 > 0, k6 > 0
    if msk is not None:
      m_s, m_d = msk & m_s, msk & m_d
      m5, m6 = msk & m5, msk & m6
    plsc.addupdate_scatter(hist_v, [(ROW_GS + cs) * NREL + t], ones16, mask=m_s)
    plsc.addupdate_scatter(hist_v, [(ROW_GD + cd) * NREL + t], ones16, mask=m_d)
    plsc.addupdate_scatter(hist_v, [(ROW_P5 + k5) * NREL + t], ones16, mask=m5)
    plsc.addupdate_scatter(hist_v, [(ROW_P6 + k6) * NREL + t], ones16, mask=m6)

  def make_ebody(bb):
    def ebody(i):
      off = bb + i * 16
      accumulate(src_v[pl.ds(off, 16)], dst_v[pl.ds(off, 16)],
                 typ_v[pl.ds(off, 16)], None)
    return ebody

  for c in range(n_chunks):
    b = c % 2
    for h in pending[b]:
      h.wait()
    bb = b * CBUF
    clen = CHUNKS[c]
    n_full = clen // 16
    plsc.parallel_loop(0, n_full, unroll=4)(make_ebody(bb))
    rem = clen - n_full * 16
    if rem:
      moff = bb + n_full * 16
      mt = lanes < rem
      accumulate(jnp.where(mt, src_v[pl.ds(moff, 16)], 0),
                 jnp.where(mt, dst_v[pl.ds(moff, 16)], 0),
                 jnp.where(mt, typ_v[pl.ds(moff, 16)], 0), mt)
    if c + 2 < n_chunks:
      pending[b] = start_chunk(c + 2)

  pltpu.sync_copy(hist_v, out_hbm.at[pl.ds(wid * HTOT, HTOT)])


def _sc_hist(ei, etype, aux):
  call = pl.kernel(
      _sc_hist_body,
      out_type=jax.ShapeDtypeStruct((NW * HTOT,), jnp.float32),
      mesh=plsc.VectorSubcoreMesh(core_axis_name="c", subcore_axis_name="s"),
      compiler_params=pltpu.CompilerParams(needs_layout_passes=False),
      scratch_types=[
          pltpu.VMEM((N_NODES,), jnp.int32),
          pltpu.VMEM((HTOT,), jnp.float32),
          pltpu.VMEM((PTAB,), jnp.int32),
          pltpu.VMEM((288,), jnp.int32),
          pltpu.VMEM((2 * CBUF,), jnp.int32),
          pltpu.VMEM((2 * CBUF,), jnp.int32),
          pltpu.VMEM((2 * CBUF,), jnp.int32),
          pltpu.SemaphoreType.DMA,
      ],
  )
  return call(ei, etype, aux)


def _combine_body(hist_ref, meta_ref, rel_emb_ref, w6_ref, b6_ref, wr_ref,
                  br_ref, wf_ref, bf_ref, out_ref):
  f32 = jnp.float32
  H = hist_ref[0:N_ROWS, :]
  for k in range(1, NW):
    H = H + hist_ref[k * N_ROWS:(k + 1) * N_ROWS, :]
  Gs = H[ROW_GS:ROW_GS + 72]
  Gd = H[ROW_GD:ROW_GD + 72]
  P5 = H[ROW_P5:ROW_P5 + 40]
  P6 = H[ROW_P6:ROW_P6 + 40]
  Emb = rel_emb_ref[...]
  dot = lambda a, b: jnp.dot(a, b, preferred_element_type=f32)
  GsE, GdE, P5E, P6E = dot(Gs, Emb), dot(Gd, Emb), dot(P5, Emb), dot(P6, Emb)
  Gs_c = jnp.sum(Gs, axis=1, keepdims=True)
  Gd_c = jnp.sum(Gd, axis=1, keepdims=True)
  P5_c = jnp.sum(P5, axis=1, keepdims=True)
  P6_c = jnp.sum(P6, axis=1, keepdims=True)
  meta = meta_ref[...]
  B = meta.shape[1]
  ch = jnp.transpose(meta[0:1, :])
  ct = jnp.transpose(meta[1:2, :])
  pr = jnp.transpose(meta[2:3, :])
  rl = jnp.transpose(meta[3:4, :])
  Oh = (lax.broadcasted_iota(jnp.int32, (B, 72), 1) == ch).astype(f32)
  Ot = (lax.broadcasted_iota(jnp.int32, (B, 72), 1) == ct).astype(f32)
  Op = (lax.broadcasted_iota(jnp.int32, (B, 40), 1) == pr).astype(f32)
  Orl = (lax.broadcasted_iota(jnp.int32, (B, NREL), 1) == rl).astype(f32)
  S1, c1 = dot(Oh, GdE), dot(Oh, Gd_c)
  S2, c2 = dot(Oh, GsE), dot(Oh, Gs_c)
  S3, c3 = dot(Ot, GdE), dot(Ot, Gd_c)
  S4, c4 = dot(Ot, GsE), dot(Ot, Gs_c)
  M5, n5 = dot(Op, P5E), dot(Op, P5_c)
  M6, n6 = dot(Op, P6E), dot(Op, P6_c)
  W6 = w6_ref[...]
  b6 = b6_ref[...]
  cat1 = dot(S1 - M6, W6[0]) + (c1 - n6) * b6[0:1, :]
  cat2 = dot(S2 - M5, W6[1]) + (c2 - n5) * b6[1:2, :]
  cat3 = dot(S3 - M5, W6[2]) + (c3 - n5) * b6[2:3, :]
  cat4 = dot(S4 - M6, W6[3]) + (c4 - n6) * b6[3:4, :]
  cat5 = dot(M5, W6[4]) + n5 * b6[4:5, :]
  cat6 = dot(M6, W6[5]) + n6 * b6[5:6, :]
  agg = cat1 + cat2 + cat3 + cat4 + cat5 + cat6
  agg = jnp.where(agg > 0, agg, 0.2 * agg)
  tr = dot(dot(Orl, Emb), wr_ref[...]) + br_ref[...]
  rep = tr + agg
  out_ref[...] = dot(rep, wf_ref[...]) + bf_ref[...]


def kernel(edge_index, edge_type, head_ids, tail_ids, rel_labels,
           rel_emb, W6, b6, Wr, br, Wf, bf):
  B = head_ids.shape[0]
  ei = edge_index.astype(jnp.int32).reshape(-1)
  et = edge_type.astype(jnp.int32)

  ids = jnp.concatenate([head_ids, tail_ids]).astype(jnp.int32)
  eq = ids[:, None] == ids[None, :]
  codes = (jnp.argmax(eq, axis=1) + 1).astype(jnp.int32)
  ch, ct = codes[:B], codes[B:]
  pkey = ch * 65 + ct
  pfirst = jnp.argmax(pkey[:, None] == pkey[None, :], axis=1)
  pr = (pfirst + 1).astype(jnp.int32)
  nid = ids.shape[0]
  fmask = (jnp.argmax(eq, axis=1) == jnp.arange(nid)).astype(jnp.int32)
  pfmask = (pfirst == jnp.arange(B)).astype(jnp.int32)
  aux = jnp.concatenate([ids, codes, pkey, pr, fmask, pfmask]).astype(jnp.int32)

  hist = _sc_hist(ei, et, aux).reshape(NW * N_ROWS, NREL)
  meta = jnp.stack([ch, ct, pr, rel_labels.astype(jnp.int32)])

  scores = pl.pallas_call(
      _combine_body,
      out_shape=jax.ShapeDtypeStruct((B, 1), jnp.float32),
  )(hist, meta, rel_emb, W6, b6, Wr, br.reshape(1, -1), Wf, bf.reshape(1, 1))
  return scores
